# baseline (speedup 1.0000x reference)
import math

import jax
import jax.numpy as jnp
from jax.experimental import pallas as pl
from jax.experimental.pallas import tpu as pltpu

_K0 = math.sqrt(2.0 / math.pi)
_K1 = 0.044715 * _K0

_MAXROWS = 2048
_NBUF = 3
_SIZES = [256, 256, 512, 1024] + [2048] * 14 + [1024, 512, 256, 256]


def _gelu(x):
    x2 = x * x
    z = x * (_K1 * x2 + _K0)
    hx = 0.5 * x
    return hx + hx * jnp.tanh(z)


def _pipeline_body(x_hbm, o_hbm, buf, in_sem, out_sem):
    n = len(_SIZES)
    offs = [0]
    for sz in _SIZES:
        offs.append(offs[-1] + sz)

    def in_copy(i, slot):
        return pltpu.make_async_copy(
            x_hbm.at[pl.ds(offs[i], _SIZES[i]), :],
            buf.at[slot, pl.ds(0, _SIZES[i]), :],
            in_sem.at[slot])

    def out_copy(i, slot):
        return pltpu.make_async_copy(
            buf.at[slot, pl.ds(0, _SIZES[i]), :],
            o_hbm.at[pl.ds(offs[i], _SIZES[i]), :],
            out_sem.at[slot])

    for j in range(_NBUF):
        in_copy(j, j).start()
    started = _NBUF

    for i in range(n):
        slot = i % _NBUF
        in_copy(i, slot).wait()
        buf[slot, pl.ds(0, _SIZES[i]), :] = _gelu(buf[slot, pl.ds(0, _SIZES[i]), :])
        out_copy(i, slot).start()
        if started < n:
            s2 = started % _NBUF
            out_copy(started - _NBUF, s2).wait()
            in_copy(started, s2).start()
            started += 1

    for j in range(n - _NBUF, n):
        out_copy(j, j % _NBUF).wait()


def kernel(x, log_tau, log_blend):
    B, T, D = x.shape
    rows = B * T
    x2 = x.reshape(rows, D)
    out = pl.pallas_call(
        _pipeline_body,
        in_specs=[pl.BlockSpec(memory_space=pltpu.MemorySpace.HBM)],
        out_specs=pl.BlockSpec(memory_space=pltpu.MemorySpace.HBM),
        out_shape=jax.ShapeDtypeStruct((rows, D), x.dtype),
        scratch_shapes=[
            pltpu.VMEM((_NBUF, _MAXROWS, D), jnp.float32),
            pltpu.SemaphoreType.DMA((_NBUF,)),
            pltpu.SemaphoreType.DMA((_NBUF,)),
        ],
        compiler_params=pltpu.CompilerParams(
            vmem_limit_bytes=100 * 1024 * 1024,
        ),
    )(x2)
    return out.reshape(B, T, D)


# speedup vs baseline: 1.0883x; 1.0883x over previous
import math

import jax
import jax.numpy as jnp
from jax.experimental import pallas as pl
from jax.experimental.pallas import tpu as pltpu

_K0 = math.sqrt(2.0 / math.pi)
_K1 = 0.044715 * _K0


def _gelu_block(x_ref, o_ref):
    x = x_ref[...]
    x2 = x * x
    z = x * (_K1 * x2 + _K0)
    hx = 0.5 * x
    o_ref[...] = hx + hx * jnp.tanh(z)


def kernel(x, log_tau, log_blend):
    B, T, D = x.shape
    rows = B * T
    x2 = x.reshape(rows, D)
    block = 1792
    out = pl.pallas_call(
        _gelu_block,
        grid=(pl.cdiv(rows, block),),
        in_specs=[pl.BlockSpec((block, D), lambda i: (i, 0))],
        out_specs=pl.BlockSpec((block, D), lambda i: (i, 0)),
        out_shape=jax.ShapeDtypeStruct((rows, D), x.dtype),
        compiler_params=pltpu.CompilerParams(
            vmem_limit_bytes=100 * 1024 * 1024,
        ),
    )(x2)
    return out.reshape(B, T, D)


# speedup vs baseline: 1.0917x; 1.0032x over previous
import math

import jax
import jax.numpy as jnp
from jax.experimental import pallas as pl
from jax.experimental.pallas import tpu as pltpu

_K0 = math.sqrt(2.0 / math.pi)
_K1 = 0.044715 * _K0


def _gelu_block(x_ref, o_ref):
    x = x_ref[...]
    x2 = x * x
    z = x * (_K1 * x2 + _K0)
    hx = 0.5 * x
    o_ref[...] = hx + hx * jnp.tanh(z)


def kernel(x, log_tau, log_blend):
    B, T, D = x.shape
    rows = B * T
    x2 = x.reshape(rows, D)
    block = 2016
    out = pl.pallas_call(
        _gelu_block,
        grid=(pl.cdiv(rows, block),),
        in_specs=[pl.BlockSpec((block, D), lambda i: (i, 0))],
        out_specs=pl.BlockSpec((block, D), lambda i: (i, 0)),
        out_shape=jax.ShapeDtypeStruct((rows, D), x.dtype),
        compiler_params=pltpu.CompilerParams(
            vmem_limit_bytes=100 * 1024 * 1024,
        ),
    )(x2)
    return out.reshape(B, T, D)
